# baseline (speedup 1.0000x reference)
import functools

import jax
import jax.numpy as jnp
from jax import lax
from jax.experimental import pallas as pl
from jax.experimental.pallas import tpu as pltpu
from jax.experimental.pallas import tpu_sc as plsc

CH = 128
L = 16


def _mm1_body(x_ref, wrel_ref, wroot_ref, b_ref, tab_ref, root_ref):
    rr = pl.program_id(1)
    xb = x_ref[...]
    tab_ref[0] = jnp.dot(xb, wrel_ref[0], preferred_element_type=jnp.float32)

    @pl.when(rr == 0)
    def _():
        root_ref[...] = (
            jnp.dot(xb, wroot_ref[...], preferred_element_type=jnp.float32)
            + b_ref[...]
        )


def _mm2_body(parts_ref, wrel_ref, wroot_ref, b_ref, tab_ref, root_ref):
    rr = pl.program_id(1)
    h = jnp.maximum(parts_ref[0] + parts_ref[1], 0.0)
    tab_ref[0] = jnp.dot(h, wrel_ref[0], preferred_element_type=jnp.float32)

    @pl.when(rr == 0)
    def _():
        root_ref[...] = (
            jnp.dot(h, wroot_ref[...], preferred_element_type=jnp.float32)
            + b_ref[...]
        )


def _add_body(parts_ref, o_ref):
    o_ref[...] = parts_ref[0] + parts_ref[1]


def _mm1(x, wrel, wroot, b, bn=400):
    n, din = x.shape
    r, _, d = wrel.shape
    return pl.pallas_call(
        _mm1_body,
        grid=(n // bn, r),
        in_specs=[
            pl.BlockSpec((bn, din), lambda i, rr: (i, 0)),
            pl.BlockSpec((1, din, d), lambda i, rr: (rr, 0, 0)),
            pl.BlockSpec((din, d), lambda i, rr: (0, 0)),
            pl.BlockSpec((1, d), lambda i, rr: (0, 0)),
        ],
        out_specs=[
            pl.BlockSpec((1, bn, d), lambda i, rr: (rr, i, 0)),
            pl.BlockSpec((bn, d), lambda i, rr: (i, 0)),
        ],
        out_shape=[
            jax.ShapeDtypeStruct((r, n, d), jnp.float32),
            jax.ShapeDtypeStruct((n, d), jnp.float32),
        ],
    )(x, wrel, wroot, b.reshape(1, d))


def _mm2(parts, wrel, wroot, b, bn=400):
    _, n, din = parts.shape
    r, _, d = wrel.shape
    return pl.pallas_call(
        _mm2_body,
        grid=(n // bn, r),
        in_specs=[
            pl.BlockSpec((2, bn, din), lambda i, rr: (0, i, 0)),
            pl.BlockSpec((1, din, d), lambda i, rr: (rr, 0, 0)),
            pl.BlockSpec((din, d), lambda i, rr: (0, 0)),
            pl.BlockSpec((1, d), lambda i, rr: (0, 0)),
        ],
        out_specs=[
            pl.BlockSpec((1, bn, d), lambda i, rr: (rr, i, 0)),
            pl.BlockSpec((bn, d), lambda i, rr: (i, 0)),
        ],
        out_shape=[
            jax.ShapeDtypeStruct((r, n, d), jnp.float32),
            jax.ShapeDtypeStruct((n, d), jnp.float32),
        ],
    )(parts, wrel, wroot, b.reshape(1, d))


def _add_parts(parts, bn=400):
    _, n, d = parts.shape
    return pl.pallas_call(
        _add_body,
        grid=(n // bn,),
        in_specs=[pl.BlockSpec((2, bn, d), lambda i: (0, i, 0))],
        out_specs=pl.BlockSpec((bn, d), lambda i: (i, 0)),
        out_shape=jax.ShapeDtypeStruct((n, d), jnp.float32),
    )(parts)


def _make_weights_kernel(e_pad, nr_pad):
    n_rows = e_pad // CH
    per_tile = n_rows // 16
    inv_per_tile = nr_pad // 16
    K = 8
    n_oct = per_tile // K
    mesh = plsc.VectorSubcoreMesh(core_axis_name="c", subcore_axis_name="s")

    @functools.partial(
        pl.kernel,
        mesh=mesh,
        out_type=jax.ShapeDtypeStruct((n_rows, CH), jnp.float32),
        compiler_params=pltpu.CompilerParams(
            needs_layout_passes=False, use_tc_tiling_on_sc=False),
        scratch_types=[
            pltpu.VMEM((per_tile, 1, CH), jnp.int32),
            pltpu.VMEM((CH,), jnp.float32),
            pltpu.VMEM((inv_per_tile,), jnp.float32),
            pltpu.VMEM((nr_pad,), jnp.float32),
            [pltpu.VMEM((CH,), jnp.float32) for _ in range(K)],
            [pltpu.SemaphoreType.DMA for _ in range(K)],
            pltpu.VMEM_SHARED((nr_pad,), jnp.float32),
        ],
    )
    def kern(g2_hbm, w_hbm, g2_v, ones_v, slice_v, inv_v, w_ring, sems,
             cnt_sh):
        cid = lax.axis_index("c")
        sid = lax.axis_index("s")

        @pl.when(cid == 0)
        def _():
            for i in range(CH // L):
                ones_v[pl.ds(i * L, L)] = jnp.full((L,), 1.0, jnp.float32)

            def zfill(j, _):
                slice_v[pl.ds(j * L, L)] = jnp.zeros((L,), jnp.float32)
                return 0
            lax.fori_loop(0, inv_per_tile // L, zfill, 0)
            cbase = sid * inv_per_tile
            pltpu.sync_copy(slice_v, cnt_sh.at[pl.ds(cbase, inv_per_tile)])

            rbase = sid * per_tile
            pltpu.sync_copy(g2_hbm.at[pl.ds(rbase, per_tile)], g2_v)
            plsc.subcore_barrier()

            for s in range(K):
                pltpu.async_copy(ones_v, cnt_sh.at[g2_v.at[s, 0]], sems[s],
                                 add=True)

            def oct_body(q, _):
                for s in range(K):
                    pltpu.make_async_copy(
                        ones_v, cnt_sh.at[g2_v.at[s, 0]], sems[s]).wait()
                    pltpu.async_copy(
                        ones_v, cnt_sh.at[g2_v.at[q * K + s, 0]], sems[s],
                        add=True)
                return 0
            lax.fori_loop(1, n_oct, oct_body, 0)
            for s in range(K):
                pltpu.make_async_copy(
                    ones_v, cnt_sh.at[g2_v.at[s, 0]], sems[s]).wait()
            plsc.subcore_barrier()

            pltpu.sync_copy(cnt_sh.at[pl.ds(cbase, inv_per_tile)], slice_v)

            def inv_body(j, _):
                c = slice_v[pl.ds(j * L, L)]
                slice_v[pl.ds(j * L, L)] = 1.0 / jnp.maximum(c, 1.0)
                return 0
            lax.fori_loop(0, inv_per_tile // L, inv_body, 0)
            pltpu.sync_copy(slice_v, cnt_sh.at[pl.ds(cbase, inv_per_tile)])
            plsc.subcore_barrier()
            pltpu.sync_copy(cnt_sh, inv_v)

            def wchunk(c, s):
                for i in range(CH // L):
                    g2 = g2_v[c, 0, pl.ds(i * L, L)]
                    w_ring[s][pl.ds(i * L, L)] = plsc.load_gather(inv_v, [g2])
                pltpu.async_copy(w_ring[s], w_hbm.at[rbase + c], sems[s])

            for s in range(K):
                wchunk(s, s)

            def woct_body(q, _):
                for s in range(K):
                    pltpu.make_async_copy(
                        w_ring[s], w_hbm.at[0], sems[s]).wait()
                    wchunk(q * K + s, s)
                return 0
            lax.fori_loop(1, n_oct, woct_body, 0)
            for s in range(K):
                pltpu.make_async_copy(w_ring[s], w_hbm.at[0], sems[s]).wait()

    return kern


def _make_edge_pass(n, d, e_pad, acc_rows, ch, K, P):
    n_rows = e_pad // ch
    per_w = n_rows // 32
    g = d // L
    n_rounds = per_w // K
    assert per_w % K == 0 and P < K
    rows_per_tile = acc_rows // 16
    full_tiles = n // rows_per_tile
    rem_rows = n - full_tiles * rows_per_tile
    zrows = min(ch, CH)
    mesh = plsc.VectorSubcoreMesh(core_axis_name="c", subcore_axis_name="s")

    @functools.partial(
        pl.kernel,
        mesh=mesh,
        out_type=jax.ShapeDtypeStruct((2, n, d), jnp.float32),
        compiler_params=pltpu.CompilerParams(
            needs_layout_passes=False, use_tc_tiling_on_sc=False),
        scratch_types=[
            pltpu.VMEM((per_w, 1, ch), jnp.int32),
            [pltpu.VMEM((1, ch), jnp.int32) for _ in range(K)],
            [pltpu.VMEM((ch,), jnp.float32) for _ in range(K)],
            [pltpu.VMEM((ch, d), jnp.float32) for _ in range(K)],
            [pltpu.SemaphoreType.DMA for _ in range(K)],
            [pltpu.SemaphoreType.DMA for _ in range(K)],
            [pltpu.SemaphoreType.DMA for _ in range(K)],
            pltpu.VMEM_SHARED((acc_rows, d), jnp.float32),
        ],
    )
    def kern(tab_hbm, root_hbm, g1_hbm, dst_hbm, w_hbm, out_hbm,
             g1_v, dst_ring, w_ring, rows, gsems, ssems, isems, acc_sh):
        cid = lax.axis_index("c")
        sid = lax.axis_index("s")
        wid = cid * 16 + sid
        rbase = wid * per_w

        pltpu.sync_copy(g1_hbm.at[pl.ds(rbase, per_w)], g1_v)

        def zrow(i, _):
            for k in range(g):
                rows[0][i, pl.ds(k * L, L)] = jnp.zeros((L,), jnp.float32)
            return 0
        lax.fori_loop(0, zrows, zrow, 0)

        base_row = sid * rows_per_tile

        @pl.when(jnp.logical_or(cid != 0, sid >= full_tiles))
        def _():
            for bidx in range(rows_per_tile // zrows):
                pltpu.sync_copy(
                    rows[0].at[pl.ds(0, zrows)],
                    acc_sh.at[pl.ds(base_row + bidx * zrows, zrows)])

        @pl.when(jnp.logical_and(cid == 0, sid < full_tiles))
        def _():
            pltpu.sync_copy(root_hbm.at[pl.ds(base_row, rows_per_tile)],
                            acc_sh.at[pl.ds(base_row, rows_per_tile)])

        if rem_rows > 0:
            @pl.when(jnp.logical_and(cid == 0, sid == full_tiles))
            def _():
                pltpu.sync_copy(root_hbm.at[pl.ds(base_row, rem_rows)],
                                acc_sh.at[pl.ds(base_row, rem_rows)])

        plsc.subcore_barrier()

        def issue_fetch(c, s):
            pltpu.async_copy(dst_hbm.at[rbase + c], dst_ring[s], isems[s])
            pltpu.async_copy(w_hbm.at[rbase + c], w_ring[s], isems[s])

        def wait_fetch(s):
            pltpu.make_async_copy(dst_hbm.at[0], dst_ring[s],
                                  isems[s]).wait()
            pltpu.make_async_copy(w_hbm.at[0], w_ring[s], isems[s]).wait()

        def issue_gather(c, s):
            pltpu.async_copy(tab_hbm.at[g1_v.at[c, 0]], rows[s], gsems[s])

        def wait_gather(s):
            pltpu.make_async_copy(tab_hbm.at[g1_v.at[0, 0]], rows[s],
                                  gsems[s]).wait()

        def issue_scatter(c, s):
            pltpu.async_copy(rows[s], acc_sh.at[dst_ring[s].at[0]], ssems[s],
                             add=True)

        def wait_scatter(s):
            pltpu.make_async_copy(rows[s], acc_sh.at[dst_ring[s].at[0]],
                                  ssems[s]).wait()

        def scale(c, s):
            def grp(i, _):
                wv = w_ring[s][pl.ds(i * L, L)]
                for j in range(L):
                    erow = i * L + j
                    w = wv[j]
                    for k in range(g):
                        rows[s][erow, pl.ds(k * L, L)] = (
                            rows[s][erow, pl.ds(k * L, L)] * w)
                return 0
            lax.fori_loop(0, ch // L, grp, 0)

        def emit(c, s, skip_wait, skip_issue):
            if not skip_issue:
                if not skip_wait:
                    wait_scatter((s + P) % K)
                issue_fetch(c + P, (s + P) % K)
                issue_gather(c + P, (s + P) % K)
            wait_gather(s)
            wait_fetch(s)
            scale(c, s)
            issue_scatter(c, s)

        for s in range(P):
            issue_fetch(s, s)
            issue_gather(s, s)
        for s in range(K):
            emit(s, s, s + P < K, False)

        def round_body(q, _):
            for s in range(K):
                emit(q * K + s, s, False, False)
            return 0
        lax.fori_loop(1, n_rounds - 1, round_body, 0)

        c0 = per_w - K
        for s in range(K):
            emit(c0 + s, s, False, s + P >= K)
        for s in range(K):
            wait_scatter(s)

        plsc.subcore_barrier()

        @pl.when(sid < full_tiles)
        def _():
            pltpu.sync_copy(acc_sh.at[pl.ds(base_row, rows_per_tile)],
                            out_hbm.at[cid, pl.ds(base_row, rows_per_tile)])

        if rem_rows > 0:
            @pl.when(sid == full_tiles)
            def _():
                pltpu.sync_copy(acc_sh.at[pl.ds(base_row, rem_rows)],
                                out_hbm.at[cid, pl.ds(base_row, rem_rows)])

    return kern


def kernel(x, edge_index, edge_type, W1_rel, W1_root, b1, W2_rel, W2_root, b2):
    n, din = x.shape
    r, _, dh = W1_rel.shape
    do = W2_rel.shape[2]
    e = edge_index.shape[1]

    quantum = 640 * CH
    e_pad = ((e + quantum - 1) // quantum) * quantum
    pad = e_pad - e
    src = edge_index[0].astype(jnp.int32)
    dst = edge_index[1].astype(jnp.int32)
    typ = edge_type.astype(jnp.int32)
    if pad:
        src = jnp.concatenate([src, jnp.zeros((pad,), jnp.int32)])
        typ = jnp.concatenate([typ, jnp.zeros((pad,), jnp.int32)])
        dst = jnp.concatenate([dst, jnp.full((pad,), n, jnp.int32)])

    base_idx = typ * n + src
    g2 = (dst * r + typ).reshape(e_pad // CH, 1, CH)

    nr_pad = (((n + 1) * r + 16 * CH - 1) // (16 * CH)) * (16 * CH)
    acc_rows = ((n + 1 + 16 * CH - 1) // (16 * CH)) * (16 * CH)

    w_edge = _make_weights_kernel(e_pad, nr_pad)(g2)

    ch1 = 64
    tab1, root1 = _mm1(x, W1_rel, W1_root, b1)
    parts1 = _make_edge_pass(n, dh, e_pad, acc_rows, ch1, 4, 3)(
        tab1.reshape(n * r, dh), root1,
        base_idx.reshape(e_pad // ch1, 1, ch1),
        dst.reshape(e_pad // ch1, 1, ch1),
        w_edge.reshape(e_pad // ch1, ch1))

    ch2 = 128
    tab2, root2 = _mm2(parts1, W2_rel, W2_root, b2)
    parts2 = _make_edge_pass(n, do, e_pad, acc_rows, ch2, 5, 4)(
        tab2.reshape(n * r, do), root2,
        base_idx.reshape(e_pad // ch2, 1, ch2),
        dst.reshape(e_pad // ch2, 1, ch2),
        w_edge.reshape(e_pad // ch2, ch2))

    return _add_parts(parts2)


# speedup vs baseline: 1.3192x; 1.3192x over previous
import functools

import numpy as np

import jax
import jax.numpy as jnp
from jax import lax
from jax.experimental import pallas as pl
from jax.experimental.pallas import tpu as pltpu
from jax.experimental.pallas import tpu_sc as plsc

CH = 128
L = 16


def _mm1_body(x_ref, wrel_ref, wroot_ref, b_ref, tab_ref, root_ref):
    rr = pl.program_id(1)
    xb = x_ref[...]
    tab_ref[0] = jnp.dot(
        xb, wrel_ref[0], preferred_element_type=jnp.float32
    ).astype(jnp.bfloat16)

    @pl.when(rr == 0)
    def _():
        root_ref[...] = (
            jnp.dot(xb, wroot_ref[...], preferred_element_type=jnp.float32)
            + b_ref[...]
        )


def _mm2_body(parts_ref, wrel_ref, wroot_ref, b_ref, tab_ref, root_ref):
    rr = pl.program_id(1)
    h = jnp.maximum(parts_ref[0] + parts_ref[1], 0.0)
    tab_ref[0] = jnp.dot(
        h, wrel_ref[0], preferred_element_type=jnp.float32
    ).astype(jnp.bfloat16)

    @pl.when(rr == 0)
    def _():
        root_ref[...] = (
            jnp.dot(h, wroot_ref[...], preferred_element_type=jnp.float32)
            + b_ref[...]
        )


def _add_body(parts_ref, o_ref):
    o_ref[...] = parts_ref[0] + parts_ref[1]


def _mm1(x, wrel, wroot, b, bn=400):
    n, din = x.shape
    r, _, d = wrel.shape
    return pl.pallas_call(
        _mm1_body,
        grid=(n // bn, r),
        in_specs=[
            pl.BlockSpec((bn, din), lambda i, rr: (i, 0)),
            pl.BlockSpec((1, din, d), lambda i, rr: (rr, 0, 0)),
            pl.BlockSpec((din, d), lambda i, rr: (0, 0)),
            pl.BlockSpec((1, d), lambda i, rr: (0, 0)),
        ],
        out_specs=[
            pl.BlockSpec((1, bn, d), lambda i, rr: (rr, i, 0)),
            pl.BlockSpec((bn, d), lambda i, rr: (i, 0)),
        ],
        out_shape=[
            jax.ShapeDtypeStruct((r, n, d), jnp.bfloat16),
            jax.ShapeDtypeStruct((n, d), jnp.float32),
        ],
    )(x, wrel, wroot, b.reshape(1, d))


def _mm2(parts, wrel, wroot, b, bn=400):
    _, n, din = parts.shape
    r, _, d = wrel.shape
    return pl.pallas_call(
        _mm2_body,
        grid=(n // bn, r),
        in_specs=[
            pl.BlockSpec((2, bn, din), lambda i, rr: (0, i, 0)),
            pl.BlockSpec((1, din, d), lambda i, rr: (rr, 0, 0)),
            pl.BlockSpec((din, d), lambda i, rr: (0, 0)),
            pl.BlockSpec((1, d), lambda i, rr: (0, 0)),
        ],
        out_specs=[
            pl.BlockSpec((1, bn, d), lambda i, rr: (rr, i, 0)),
            pl.BlockSpec((bn, d), lambda i, rr: (i, 0)),
        ],
        out_shape=[
            jax.ShapeDtypeStruct((r, n, d), jnp.bfloat16),
            jax.ShapeDtypeStruct((n, d), jnp.float32),
        ],
    )(parts, wrel, wroot, b.reshape(1, d))


def _add_parts(parts, bn=400):
    _, n, d = parts.shape
    return pl.pallas_call(
        _add_body,
        grid=(n // bn,),
        in_specs=[pl.BlockSpec((2, bn, d), lambda i: (0, i, 0))],
        out_specs=pl.BlockSpec((bn, d), lambda i: (i, 0)),
        out_shape=jax.ShapeDtypeStruct((n, d), jnp.float32),
    )(parts)


def _make_weights_kernel(e_pad, nr_pad):
    n_rows = e_pad // CH
    per_tile = n_rows // 16
    inv_per_tile = nr_pad // 16
    K = 8
    n_oct = per_tile // K
    mesh = plsc.VectorSubcoreMesh(core_axis_name="c", subcore_axis_name="s")

    @functools.partial(
        pl.kernel,
        mesh=mesh,
        out_type=jax.ShapeDtypeStruct((n_rows, CH), jnp.float32),
        compiler_params=pltpu.CompilerParams(
            needs_layout_passes=False, use_tc_tiling_on_sc=False),
        scratch_types=[
            pltpu.VMEM((per_tile, 1, CH), jnp.int32),
            pltpu.VMEM((CH,), jnp.float32),
            pltpu.VMEM((inv_per_tile,), jnp.float32),
            pltpu.VMEM((nr_pad,), jnp.float32),
            [pltpu.VMEM((CH,), jnp.float32) for _ in range(K)],
            [pltpu.SemaphoreType.DMA for _ in range(K)],
            pltpu.VMEM_SHARED((nr_pad,), jnp.float32),
        ],
    )
    def kern(g2_hbm, w_hbm, g2_v, ones_v, slice_v, inv_v, w_ring, sems,
             cnt_sh):
        cid = lax.axis_index("c")
        sid = lax.axis_index("s")

        @pl.when(cid == 0)
        def _():
            for i in range(CH // L):
                ones_v[pl.ds(i * L, L)] = jnp.full((L,), 1.0, jnp.float32)

            def zfill(j, _):
                slice_v[pl.ds(j * L, L)] = jnp.zeros((L,), jnp.float32)
                return 0
            lax.fori_loop(0, inv_per_tile // L, zfill, 0)
            cbase = sid * inv_per_tile
            pltpu.sync_copy(slice_v, cnt_sh.at[pl.ds(cbase, inv_per_tile)])

            rbase = sid * per_tile
            pltpu.sync_copy(g2_hbm.at[pl.ds(rbase, per_tile)], g2_v)
            plsc.subcore_barrier()

            for s in range(K):
                pltpu.async_copy(ones_v, cnt_sh.at[g2_v.at[s, 0]], sems[s],
                                 add=True)

            def oct_body(q, _):
                for s in range(K):
                    pltpu.make_async_copy(
                        ones_v, cnt_sh.at[g2_v.at[s, 0]], sems[s]).wait()
                    pltpu.async_copy(
                        ones_v, cnt_sh.at[g2_v.at[q * K + s, 0]], sems[s],
                        add=True)
                return 0
            lax.fori_loop(1, n_oct, oct_body, 0)
            for s in range(K):
                pltpu.make_async_copy(
                    ones_v, cnt_sh.at[g2_v.at[s, 0]], sems[s]).wait()
            plsc.subcore_barrier()

            pltpu.sync_copy(cnt_sh.at[pl.ds(cbase, inv_per_tile)], slice_v)

            def inv_body(j, _):
                c = slice_v[pl.ds(j * L, L)]
                slice_v[pl.ds(j * L, L)] = 1.0 / jnp.maximum(c, 1.0)
                return 0
            lax.fori_loop(0, inv_per_tile // L, inv_body, 0)
            pltpu.sync_copy(slice_v, cnt_sh.at[pl.ds(cbase, inv_per_tile)])
            plsc.subcore_barrier()
            pltpu.sync_copy(cnt_sh, inv_v)

            def wchunk(c, s):
                for i in range(CH // L):
                    g2 = g2_v[c, 0, pl.ds(i * L, L)]
                    w_ring[s][pl.ds(i * L, L)] = plsc.load_gather(inv_v, [g2])
                pltpu.async_copy(w_ring[s], w_hbm.at[rbase + c], sems[s])

            for s in range(K):
                wchunk(s, s)

            def woct_body(q, _):
                for s in range(K):
                    pltpu.make_async_copy(
                        w_ring[s], w_hbm.at[0], sems[s]).wait()
                    wchunk(q * K + s, s)
                return 0
            lax.fori_loop(1, n_oct, woct_body, 0)
            for s in range(K):
                pltpu.make_async_copy(w_ring[s], w_hbm.at[0], sems[s]).wait()

    return kern


def _make_edge_pass(n, d, e_pad, acc_rows, ch, K, P):
    n_rows = e_pad // ch
    per_w = n_rows // 32
    g = d // L
    n_rounds = per_w // K
    assert per_w % K == 0 and P < K and K % 2 == 0 and (d // L) % 2 == 0
    rows_per_tile = acc_rows // 16
    full_tiles = n // rows_per_tile
    rem_rows = n - full_tiles * rows_per_tile
    zrows = min(ch, CH)
    mesh = plsc.VectorSubcoreMesh(core_axis_name="c", subcore_axis_name="s")

    @functools.partial(
        pl.kernel,
        mesh=mesh,
        out_type=jax.ShapeDtypeStruct((2, n, d), jnp.float32),
        compiler_params=pltpu.CompilerParams(
            needs_layout_passes=False, use_tc_tiling_on_sc=False),
        scratch_types=[
            pltpu.VMEM((per_w, 1, ch), jnp.int32),
            [pltpu.VMEM((1, ch), jnp.int32) for _ in range(K)],
            [pltpu.VMEM((ch,), jnp.float32) for _ in range(K)],
            [pltpu.VMEM((ch, d), jnp.bfloat16) for _ in range(K)],
            [pltpu.VMEM((ch, d), jnp.float32) for _ in range(2)],
            [pltpu.SemaphoreType.DMA for _ in range(K)],
            [pltpu.SemaphoreType.DMA for _ in range(2)],
            [pltpu.SemaphoreType.DMA for _ in range(K)],
            pltpu.VMEM_SHARED((acc_rows, d), jnp.float32),
        ],
    )
    def kern(tab_hbm, root_hbm, g1_hbm, dst_hbm, w_hbm, out_hbm,
             g1_v, dst_ring, w_ring, rows, stag, gsems, ssems, isems,
             acc_sh):
        cid = lax.axis_index("c")
        sid = lax.axis_index("s")
        wid = cid * 16 + sid
        rbase = wid * per_w

        pltpu.sync_copy(g1_hbm.at[pl.ds(rbase, per_w)], g1_v)

        def zrow(i, _):
            for k in range(g):
                stag[0][i, pl.ds(k * L, L)] = jnp.zeros((L,), jnp.float32)
            return 0
        lax.fori_loop(0, zrows, zrow, 0)

        base_row = sid * rows_per_tile

        @pl.when(jnp.logical_or(cid != 0, sid >= full_tiles))
        def _():
            for bidx in range(rows_per_tile // zrows):
                pltpu.sync_copy(
                    stag[0].at[pl.ds(0, zrows)],
                    acc_sh.at[pl.ds(base_row + bidx * zrows, zrows)])

        @pl.when(jnp.logical_and(cid == 0, sid < full_tiles))
        def _():
            pltpu.sync_copy(root_hbm.at[pl.ds(base_row, rows_per_tile)],
                            acc_sh.at[pl.ds(base_row, rows_per_tile)])

        if rem_rows > 0:
            @pl.when(jnp.logical_and(cid == 0, sid == full_tiles))
            def _():
                pltpu.sync_copy(root_hbm.at[pl.ds(base_row, rem_rows)],
                                acc_sh.at[pl.ds(base_row, rem_rows)])

        plsc.subcore_barrier()

        def issue_fetch(c, s):
            pltpu.async_copy(dst_hbm.at[rbase + c], dst_ring[s], isems[s])
            pltpu.async_copy(w_hbm.at[rbase + c], w_ring[s], isems[s])

        def wait_fetch(s):
            pltpu.make_async_copy(dst_hbm.at[0], dst_ring[s],
                                  isems[s]).wait()
            pltpu.make_async_copy(w_hbm.at[0], w_ring[s], isems[s]).wait()

        def issue_gather(c, s):
            pltpu.async_copy(tab_hbm.at[g1_v.at[c, 0]], rows[s], gsems[s])

        def wait_gather(s):
            pltpu.make_async_copy(tab_hbm.at[g1_v.at[0, 0]], rows[s],
                                  gsems[s]).wait()

        def issue_scatter(s, t):
            pltpu.async_copy(stag[t], acc_sh.at[dst_ring[s].at[0]], ssems[t],
                             add=True)

        def wait_scatter(s, t):
            pltpu.make_async_copy(stag[t], acc_sh.at[dst_ring[s].at[0]],
                                  ssems[t]).wait()

        def scale(s, t):
            def grp(i, _):
                wv = w_ring[s][pl.ds(i * L, L)]
                for j in range(L):
                    erow = i * L + j
                    w = wv[j]
                    for k in range(g // 2):
                        mb = rows[s][erow, pl.ds(k * 2 * L, 2 * L)]
                        a, b2 = plsc.unpack(
                            mb, format=plsc.PackFormat.INTERLEAVED)
                        stag[t][erow, pl.ds(k * 2 * L, L)] = a * w
                        stag[t][erow, pl.ds(k * 2 * L + L, L)] = b2 * w
                return 0
            lax.fori_loop(0, ch // L, grp, 0)

        def emit(c, s, t, first, skip_issue):
            if not first:
                wait_scatter((s + K - 1) % K, 1 - t)
            if not skip_issue:
                issue_fetch(c + P, (s + P) % K)
                issue_gather(c + P, (s + P) % K)
            wait_gather(s)
            wait_fetch(s)
            scale(s, t)
            issue_scatter(s, t)

        for s in range(P):
            issue_fetch(s, s)
            issue_gather(s, s)
        for s in range(K):
            emit(s, s, s % 2, s == 0, False)

        def round_body(q, _):
            for s in range(K):
                emit(q * K + s, s, s % 2, False, False)
            return 0
        lax.fori_loop(1, n_rounds - 1, round_body, 0)

        c0 = per_w - K
        for s in range(K):
            emit(c0 + s, s, s % 2, False, s + P >= K)
        wait_scatter(K - 1, (K - 1) % 2)

        plsc.subcore_barrier()

        @pl.when(sid < full_tiles)
        def _():
            pltpu.sync_copy(acc_sh.at[pl.ds(base_row, rows_per_tile)],
                            out_hbm.at[cid, pl.ds(base_row, rows_per_tile)])

        if rem_rows > 0:
            @pl.when(sid == full_tiles)
            def _():
                pltpu.sync_copy(acc_sh.at[pl.ds(base_row, rem_rows)],
                                out_hbm.at[cid, pl.ds(base_row, rem_rows)])

    return kern


def kernel(x, edge_index, edge_type, W1_rel, W1_root, b1, W2_rel, W2_root, b2):
    n, din = x.shape
    r, _, dh = W1_rel.shape
    do = W2_rel.shape[2]
    e = edge_index.shape[1]

    quantum = 640 * CH
    e_pad = ((e + quantum - 1) // quantum) * quantum
    pad = e_pad - e
    src = edge_index[0].astype(jnp.int32)
    dst = edge_index[1].astype(jnp.int32)
    typ = edge_type.astype(jnp.int32)
    if pad:
        src = jnp.concatenate([src, jnp.zeros((pad,), jnp.int32)])
        typ = jnp.concatenate([typ, jnp.zeros((pad,), jnp.int32)])
        dst = jnp.concatenate([dst, jnp.full((pad,), n, jnp.int32)])

    base_idx = typ * n + src
    g2 = (dst * r + typ).reshape(e_pad // CH, 1, CH)

    nr_pad = (((n + 1) * r + 16 * CH - 1) // (16 * CH)) * (16 * CH)
    acc_rows = ((n + 1 + 16 * CH - 1) // (16 * CH)) * (16 * CH)

    w_edge = _make_weights_kernel(e_pad, nr_pad)(g2)

    def _ileave(dd):
        p = []
        for b0 in range(0, dd, 2 * L):
            for k in range(L):
                p.extend([b0 + k, b0 + L + k])
        return np.array(p)

    w1p = W1_rel[:, :, _ileave(dh)]
    w2p = W2_rel[:, :, _ileave(do)]

    ch1 = 64
    tab1, root1 = _mm1(x, w1p, W1_root, b1)
    parts1 = _make_edge_pass(n, dh, e_pad, acc_rows, ch1, 4, 3)(
        tab1.reshape(n * r, dh), root1,
        base_idx.reshape(e_pad // ch1, 1, ch1),
        dst.reshape(e_pad // ch1, 1, ch1),
        w_edge.reshape(e_pad // ch1, ch1))

    ch2 = 128
    tab2, root2 = _mm2(parts1, w2p, W2_root, b2)
    parts2 = _make_edge_pass(n, do, e_pad, acc_rows, ch2, 4, 3)(
        tab2.reshape(n * r, do), root2,
        base_idx.reshape(e_pad // ch2, 1, ch2),
        dst.reshape(e_pad // ch2, 1, ch2),
        w_edge.reshape(e_pad // ch2, ch2))

    return _add_parts(parts2)


# speedup vs baseline: 1.3195x; 1.0002x over previous
import functools

import numpy as np

import jax
import jax.numpy as jnp
from jax import lax
from jax.experimental import pallas as pl
from jax.experimental.pallas import tpu as pltpu
from jax.experimental.pallas import tpu_sc as plsc

CH = 128
L = 16


def _mm1_body(x_ref, wrel_ref, wroot_ref, b_ref, tab_ref, root_ref):
    rr = pl.program_id(1)
    xb = x_ref[...]
    tab_ref[0] = jnp.dot(
        xb, wrel_ref[0], preferred_element_type=jnp.float32
    ).astype(jnp.bfloat16)

    @pl.when(rr == 0)
    def _():
        root_ref[...] = (
            jnp.dot(xb, wroot_ref[...], preferred_element_type=jnp.float32)
            + b_ref[...]
        )


def _mm2_body(parts_ref, wrel_ref, wroot_ref, b_ref, tab_ref, root_ref):
    rr = pl.program_id(1)
    h = jnp.maximum(parts_ref[0] + parts_ref[1], 0.0)
    tab_ref[0] = jnp.dot(
        h, wrel_ref[0], preferred_element_type=jnp.float32
    ).astype(jnp.bfloat16)

    @pl.when(rr == 0)
    def _():
        root_ref[...] = (
            jnp.dot(h, wroot_ref[...], preferred_element_type=jnp.float32)
            + b_ref[...]
        )


def _add_body(parts_ref, o_ref):
    o_ref[...] = parts_ref[0] + parts_ref[1]


def _mm1(x, wrel, wroot, b, bn=400):
    n, din = x.shape
    r, _, d = wrel.shape
    return pl.pallas_call(
        _mm1_body,
        grid=(n // bn, r),
        in_specs=[
            pl.BlockSpec((bn, din), lambda i, rr: (i, 0)),
            pl.BlockSpec((1, din, d), lambda i, rr: (rr, 0, 0)),
            pl.BlockSpec((din, d), lambda i, rr: (0, 0)),
            pl.BlockSpec((1, d), lambda i, rr: (0, 0)),
        ],
        out_specs=[
            pl.BlockSpec((1, bn, d), lambda i, rr: (rr, i, 0)),
            pl.BlockSpec((bn, d), lambda i, rr: (i, 0)),
        ],
        out_shape=[
            jax.ShapeDtypeStruct((r, n, d), jnp.bfloat16),
            jax.ShapeDtypeStruct((n, d), jnp.float32),
        ],
    )(x, wrel, wroot, b.reshape(1, d))


def _mm2(parts, wrel, wroot, b, bn=400):
    _, n, din = parts.shape
    r, _, d = wrel.shape
    return pl.pallas_call(
        _mm2_body,
        grid=(n // bn, r),
        in_specs=[
            pl.BlockSpec((2, bn, din), lambda i, rr: (0, i, 0)),
            pl.BlockSpec((1, din, d), lambda i, rr: (rr, 0, 0)),
            pl.BlockSpec((din, d), lambda i, rr: (0, 0)),
            pl.BlockSpec((1, d), lambda i, rr: (0, 0)),
        ],
        out_specs=[
            pl.BlockSpec((1, bn, d), lambda i, rr: (rr, i, 0)),
            pl.BlockSpec((bn, d), lambda i, rr: (i, 0)),
        ],
        out_shape=[
            jax.ShapeDtypeStruct((r, n, d), jnp.bfloat16),
            jax.ShapeDtypeStruct((n, d), jnp.float32),
        ],
    )(parts, wrel, wroot, b.reshape(1, d))


def _add_parts(parts, bn=400):
    _, n, d = parts.shape
    return pl.pallas_call(
        _add_body,
        grid=(n // bn,),
        in_specs=[pl.BlockSpec((2, bn, d), lambda i: (0, i, 0))],
        out_specs=pl.BlockSpec((bn, d), lambda i: (i, 0)),
        out_shape=jax.ShapeDtypeStruct((n, d), jnp.float32),
    )(parts)


def _make_weights_kernel(e_pad, nr_pad):
    n_rows = e_pad // CH
    per_tile = n_rows // 16
    inv_per_tile = nr_pad // 16
    K = 8
    n_oct = per_tile // K
    mesh = plsc.VectorSubcoreMesh(core_axis_name="c", subcore_axis_name="s")

    @functools.partial(
        pl.kernel,
        mesh=mesh,
        out_type=jax.ShapeDtypeStruct((n_rows, CH), jnp.float32),
        compiler_params=pltpu.CompilerParams(
            needs_layout_passes=False, use_tc_tiling_on_sc=False),
        scratch_types=[
            pltpu.VMEM((per_tile, 1, CH), jnp.int32),
            pltpu.VMEM((CH,), jnp.float32),
            pltpu.VMEM((inv_per_tile,), jnp.float32),
            pltpu.VMEM((nr_pad,), jnp.float32),
            [pltpu.VMEM((CH,), jnp.float32) for _ in range(K)],
            [pltpu.SemaphoreType.DMA for _ in range(K)],
            pltpu.VMEM_SHARED((nr_pad,), jnp.float32),
        ],
    )
    def kern(g2_hbm, w_hbm, g2_v, ones_v, slice_v, inv_v, w_ring, sems,
             cnt_sh):
        cid = lax.axis_index("c")
        sid = lax.axis_index("s")

        @pl.when(cid == 0)
        def _():
            for i in range(CH // L):
                ones_v[pl.ds(i * L, L)] = jnp.full((L,), 1.0, jnp.float32)

            def zfill(j, _):
                slice_v[pl.ds(j * L, L)] = jnp.zeros((L,), jnp.float32)
                return 0
            lax.fori_loop(0, inv_per_tile // L, zfill, 0)
            cbase = sid * inv_per_tile
            pltpu.sync_copy(slice_v, cnt_sh.at[pl.ds(cbase, inv_per_tile)])

            rbase = sid * per_tile
            pltpu.sync_copy(g2_hbm.at[pl.ds(rbase, per_tile)], g2_v)
            plsc.subcore_barrier()

            for s in range(K):
                pltpu.async_copy(ones_v, cnt_sh.at[g2_v.at[s, 0]], sems[s],
                                 add=True)

            def oct_body(q, _):
                for s in range(K):
                    pltpu.make_async_copy(
                        ones_v, cnt_sh.at[g2_v.at[s, 0]], sems[s]).wait()
                    pltpu.async_copy(
                        ones_v, cnt_sh.at[g2_v.at[q * K + s, 0]], sems[s],
                        add=True)
                return 0
            lax.fori_loop(1, n_oct, oct_body, 0)
            for s in range(K):
                pltpu.make_async_copy(
                    ones_v, cnt_sh.at[g2_v.at[s, 0]], sems[s]).wait()
            plsc.subcore_barrier()

            pltpu.sync_copy(cnt_sh.at[pl.ds(cbase, inv_per_tile)], slice_v)

            def inv_body(j, _):
                c = slice_v[pl.ds(j * L, L)]
                slice_v[pl.ds(j * L, L)] = 1.0 / jnp.maximum(c, 1.0)
                return 0
            lax.fori_loop(0, inv_per_tile // L, inv_body, 0)
            pltpu.sync_copy(slice_v, cnt_sh.at[pl.ds(cbase, inv_per_tile)])
            plsc.subcore_barrier()
            pltpu.sync_copy(cnt_sh, inv_v)

            def wchunk(c, s):
                for i in range(CH // L):
                    g2 = g2_v[c, 0, pl.ds(i * L, L)]
                    w_ring[s][pl.ds(i * L, L)] = plsc.load_gather(inv_v, [g2])
                pltpu.async_copy(w_ring[s], w_hbm.at[rbase + c], sems[s])

            for s in range(K):
                wchunk(s, s)

            def woct_body(q, _):
                for s in range(K):
                    pltpu.make_async_copy(
                        w_ring[s], w_hbm.at[0], sems[s]).wait()
                    wchunk(q * K + s, s)
                return 0
            lax.fori_loop(1, n_oct, woct_body, 0)
            for s in range(K):
                pltpu.make_async_copy(w_ring[s], w_hbm.at[0], sems[s]).wait()

    return kern


def _make_edge_pass(n, d, e_pad, acc_rows, ch, K, P):
    n_rows = e_pad // ch
    per_w = n_rows // 32
    g = d // L
    n_rounds = per_w // K
    assert per_w % K == 0 and P < K and K % 2 == 0 and (d // L) % 2 == 0
    rows_per_tile = acc_rows // 16
    full_tiles = n // rows_per_tile
    rem_rows = n - full_tiles * rows_per_tile
    zrows = min(ch, CH)
    mesh = plsc.VectorSubcoreMesh(core_axis_name="c", subcore_axis_name="s")

    @functools.partial(
        pl.kernel,
        mesh=mesh,
        out_type=jax.ShapeDtypeStruct((2, n, d), jnp.float32),
        compiler_params=pltpu.CompilerParams(
            needs_layout_passes=False, use_tc_tiling_on_sc=False),
        scratch_types=[
            pltpu.VMEM((per_w, 1, ch), jnp.int32),
            [pltpu.VMEM((1, ch), jnp.int32) for _ in range(K)],
            [pltpu.VMEM((ch,), jnp.float32) for _ in range(K)],
            [pltpu.VMEM((ch, d), jnp.bfloat16) for _ in range(K)],
            [pltpu.VMEM((ch, d), jnp.float32) for _ in range(2)],
            [pltpu.SemaphoreType.DMA for _ in range(K)],
            [pltpu.SemaphoreType.DMA for _ in range(K)],
            [pltpu.SemaphoreType.DMA for _ in range(2)],
            [pltpu.SemaphoreType.DMA for _ in range(K)],
            pltpu.VMEM_SHARED((acc_rows, d), jnp.float32),
        ],
    )
    def kern(tab_hbm, root_hbm, g1_hbm, dst_hbm, w_hbm, out_hbm,
             g1_v, dst_ring, w_ring, rows, stag, gsems, gsems2, ssems,
             isems, acc_sh):
        cid = lax.axis_index("c")
        sid = lax.axis_index("s")
        wid = cid * 16 + sid
        rbase = wid * per_w

        pltpu.sync_copy(g1_hbm.at[pl.ds(rbase, per_w)], g1_v)

        def zrow(i, _):
            for k in range(g):
                stag[0][i, pl.ds(k * L, L)] = jnp.zeros((L,), jnp.float32)
            return 0
        lax.fori_loop(0, zrows, zrow, 0)

        base_row = sid * rows_per_tile

        @pl.when(jnp.logical_or(cid != 0, sid >= full_tiles))
        def _():
            for bidx in range(rows_per_tile // zrows):
                pltpu.sync_copy(
                    stag[0].at[pl.ds(0, zrows)],
                    acc_sh.at[pl.ds(base_row + bidx * zrows, zrows)])

        @pl.when(jnp.logical_and(cid == 0, sid < full_tiles))
        def _():
            pltpu.sync_copy(root_hbm.at[pl.ds(base_row, rows_per_tile)],
                            acc_sh.at[pl.ds(base_row, rows_per_tile)])

        if rem_rows > 0:
            @pl.when(jnp.logical_and(cid == 0, sid == full_tiles))
            def _():
                pltpu.sync_copy(root_hbm.at[pl.ds(base_row, rem_rows)],
                                acc_sh.at[pl.ds(base_row, rem_rows)])

        plsc.subcore_barrier()

        def issue_fetch(c, s):
            pltpu.async_copy(dst_hbm.at[rbase + c], dst_ring[s], isems[s])
            pltpu.async_copy(w_hbm.at[rbase + c], w_ring[s], isems[s])

        def wait_fetch(s):
            pltpu.make_async_copy(dst_hbm.at[0], dst_ring[s],
                                  isems[s]).wait()
            pltpu.make_async_copy(w_hbm.at[0], w_ring[s], isems[s]).wait()

        def issue_gather(c, s):
            h2 = ch // 2
            pltpu.async_copy(tab_hbm.at[g1_v.at[c, 0, pl.ds(0, h2)]],
                             rows[s].at[pl.ds(0, h2)], gsems[s])
            pltpu.async_copy(tab_hbm.at[g1_v.at[c, 0, pl.ds(h2, h2)]],
                             rows[s].at[pl.ds(h2, h2)], gsems2[s])

        def wait_gather(s):
            h2 = ch // 2
            pltpu.make_async_copy(tab_hbm.at[g1_v.at[0, 0, pl.ds(0, h2)]],
                                  rows[s].at[pl.ds(0, h2)], gsems[s]).wait()
            pltpu.make_async_copy(tab_hbm.at[g1_v.at[0, 0, pl.ds(0, h2)]],
                                  rows[s].at[pl.ds(h2, h2)], gsems2[s]).wait()

        def issue_scatter(s, t):
            pltpu.async_copy(stag[t], acc_sh.at[dst_ring[s].at[0]], ssems[t],
                             add=True)

        def wait_scatter(s, t):
            pltpu.make_async_copy(stag[t], acc_sh.at[dst_ring[s].at[0]],
                                  ssems[t]).wait()

        def scale(s, t):
            def grp(i, _):
                wv = w_ring[s][pl.ds(i * L, L)]
                for j in range(L):
                    erow = i * L + j
                    w = wv[j]
                    for k in range(g // 2):
                        mb = rows[s][erow, pl.ds(k * 2 * L, 2 * L)]
                        a, b2 = plsc.unpack(
                            mb, format=plsc.PackFormat.INTERLEAVED)
                        stag[t][erow, pl.ds(k * 2 * L, L)] = a * w
                        stag[t][erow, pl.ds(k * 2 * L + L, L)] = b2 * w
                return 0
            lax.fori_loop(0, ch // L, grp, 0)

        def emit(c, s, t, first, skip_issue):
            if not first:
                wait_scatter((s + K - 1) % K, 1 - t)
            if not skip_issue:
                issue_fetch(c + P, (s + P) % K)
                issue_gather(c + P, (s + P) % K)
            wait_gather(s)
            wait_fetch(s)
            scale(s, t)
            issue_scatter(s, t)

        for s in range(P):
            issue_fetch(s, s)
            issue_gather(s, s)
        for s in range(K):
            emit(s, s, s % 2, s == 0, False)

        def round_body(q, _):
            for s in range(K):
                emit(q * K + s, s, s % 2, False, False)
            return 0
        lax.fori_loop(1, n_rounds - 1, round_body, 0)

        c0 = per_w - K
        for s in range(K):
            emit(c0 + s, s, s % 2, False, s + P >= K)
        wait_scatter(K - 1, (K - 1) % 2)

        plsc.subcore_barrier()

        @pl.when(sid < full_tiles)
        def _():
            pltpu.sync_copy(acc_sh.at[pl.ds(base_row, rows_per_tile)],
                            out_hbm.at[cid, pl.ds(base_row, rows_per_tile)])

        if rem_rows > 0:
            @pl.when(sid == full_tiles)
            def _():
                pltpu.sync_copy(acc_sh.at[pl.ds(base_row, rem_rows)],
                                out_hbm.at[cid, pl.ds(base_row, rem_rows)])

    return kern


def kernel(x, edge_index, edge_type, W1_rel, W1_root, b1, W2_rel, W2_root, b2):
    n, din = x.shape
    r, _, dh = W1_rel.shape
    do = W2_rel.shape[2]
    e = edge_index.shape[1]

    quantum = 640 * CH
    e_pad = ((e + quantum - 1) // quantum) * quantum
    pad = e_pad - e
    src = edge_index[0].astype(jnp.int32)
    dst = edge_index[1].astype(jnp.int32)
    typ = edge_type.astype(jnp.int32)
    if pad:
        src = jnp.concatenate([src, jnp.zeros((pad,), jnp.int32)])
        typ = jnp.concatenate([typ, jnp.zeros((pad,), jnp.int32)])
        dst = jnp.concatenate([dst, jnp.full((pad,), n, jnp.int32)])

    base_idx = typ * n + src
    g2 = (dst * r + typ).reshape(e_pad // CH, 1, CH)

    nr_pad = (((n + 1) * r + 16 * CH - 1) // (16 * CH)) * (16 * CH)
    acc_rows = ((n + 1 + 16 * CH - 1) // (16 * CH)) * (16 * CH)

    w_edge = _make_weights_kernel(e_pad, nr_pad)(g2)

    def _ileave(dd):
        p = []
        for b0 in range(0, dd, 2 * L):
            for k in range(L):
                p.extend([b0 + k, b0 + L + k])
        return np.array(p)

    w1p = W1_rel[:, :, _ileave(dh)]
    w2p = W2_rel[:, :, _ileave(do)]

    ch1 = 64
    tab1, root1 = _mm1(x, w1p, W1_root, b1)
    parts1 = _make_edge_pass(n, dh, e_pad, acc_rows, ch1, 4, 3)(
        tab1.reshape(n * r, dh), root1,
        base_idx.reshape(e_pad // ch1, 1, ch1),
        dst.reshape(e_pad // ch1, 1, ch1),
        w_edge.reshape(e_pad // ch1, ch1))

    ch2 = 128
    tab2, root2 = _mm2(parts1, w2p, W2_root, b2)
    parts2 = _make_edge_pass(n, do, e_pad, acc_rows, ch2, 4, 3)(
        tab2.reshape(n * r, do), root2,
        base_idx.reshape(e_pad // ch2, 1, ch2),
        dst.reshape(e_pad // ch2, 1, ch2),
        w_edge.reshape(e_pad // ch2, ch2))

    return _add_parts(parts2)


# speedup vs baseline: 1.3195x; 1.0000x over previous
import functools

import numpy as np

import jax
import jax.numpy as jnp
from jax import lax
from jax.experimental import pallas as pl
from jax.experimental.pallas import tpu as pltpu
from jax.experimental.pallas import tpu_sc as plsc

CH = 128
L = 16


def _mm1_body(x_ref, wrel_ref, wroot_ref, b_ref, tab_ref, root_ref):
    rr = pl.program_id(1)
    xb = x_ref[...]
    tab_ref[0] = jnp.dot(
        xb, wrel_ref[0], preferred_element_type=jnp.float32
    ).astype(jnp.bfloat16)

    @pl.when(rr == 0)
    def _():
        root_ref[...] = (
            jnp.dot(xb, wroot_ref[...], preferred_element_type=jnp.float32)
            + b_ref[...]
        )


def _mm2_body(parts_ref, wrel_ref, wroot_ref, b_ref, tab_ref, root_ref):
    rr = pl.program_id(1)
    h = jnp.maximum(parts_ref[0] + parts_ref[1], 0.0)
    tab_ref[0] = jnp.dot(
        h, wrel_ref[0], preferred_element_type=jnp.float32
    ).astype(jnp.bfloat16)

    @pl.when(rr == 0)
    def _():
        root_ref[...] = (
            jnp.dot(h, wroot_ref[...], preferred_element_type=jnp.float32)
            + b_ref[...]
        )


def _add_body(parts_ref, o_ref):
    o_ref[...] = parts_ref[0] + parts_ref[1]


def _mm1(x, wrel, wroot, b, bn=400):
    n, din = x.shape
    r, _, d = wrel.shape
    return pl.pallas_call(
        _mm1_body,
        grid=(n // bn, r),
        in_specs=[
            pl.BlockSpec((bn, din), lambda i, rr: (i, 0)),
            pl.BlockSpec((1, din, d), lambda i, rr: (rr, 0, 0)),
            pl.BlockSpec((din, d), lambda i, rr: (0, 0)),
            pl.BlockSpec((1, d), lambda i, rr: (0, 0)),
        ],
        out_specs=[
            pl.BlockSpec((1, bn, d), lambda i, rr: (rr, i, 0)),
            pl.BlockSpec((bn, d), lambda i, rr: (i, 0)),
        ],
        out_shape=[
            jax.ShapeDtypeStruct((r, n, d), jnp.bfloat16),
            jax.ShapeDtypeStruct((n, d), jnp.float32),
        ],
    )(x, wrel, wroot, b.reshape(1, d))


def _mm2(parts, wrel, wroot, b, bn=400):
    _, n, din = parts.shape
    r, _, d = wrel.shape
    return pl.pallas_call(
        _mm2_body,
        grid=(n // bn, r),
        in_specs=[
            pl.BlockSpec((2, bn, din), lambda i, rr: (0, i, 0)),
            pl.BlockSpec((1, din, d), lambda i, rr: (rr, 0, 0)),
            pl.BlockSpec((din, d), lambda i, rr: (0, 0)),
            pl.BlockSpec((1, d), lambda i, rr: (0, 0)),
        ],
        out_specs=[
            pl.BlockSpec((1, bn, d), lambda i, rr: (rr, i, 0)),
            pl.BlockSpec((bn, d), lambda i, rr: (i, 0)),
        ],
        out_shape=[
            jax.ShapeDtypeStruct((r, n, d), jnp.bfloat16),
            jax.ShapeDtypeStruct((n, d), jnp.float32),
        ],
    )(parts, wrel, wroot, b.reshape(1, d))


def _add_parts(parts, bn=400):
    _, n, d = parts.shape
    return pl.pallas_call(
        _add_body,
        grid=(n // bn,),
        in_specs=[pl.BlockSpec((2, bn, d), lambda i: (0, i, 0))],
        out_specs=pl.BlockSpec((bn, d), lambda i: (i, 0)),
        out_shape=jax.ShapeDtypeStruct((n, d), jnp.float32),
    )(parts)


def _make_weights_kernel(e_pad, nr_pad):
    n_rows = e_pad // CH
    per_tile = n_rows // 16
    inv_per_tile = nr_pad // 16
    K = 8
    n_oct = per_tile // K
    mesh = plsc.VectorSubcoreMesh(core_axis_name="c", subcore_axis_name="s")

    @functools.partial(
        pl.kernel,
        mesh=mesh,
        out_type=jax.ShapeDtypeStruct((n_rows, CH), jnp.float32),
        compiler_params=pltpu.CompilerParams(
            needs_layout_passes=False, use_tc_tiling_on_sc=False),
        scratch_types=[
            pltpu.VMEM((per_tile, 1, CH), jnp.int32),
            pltpu.VMEM((CH,), jnp.float32),
            pltpu.VMEM((inv_per_tile,), jnp.float32),
            pltpu.VMEM((nr_pad,), jnp.float32),
            [pltpu.VMEM((CH,), jnp.float32) for _ in range(K)],
            [pltpu.SemaphoreType.DMA for _ in range(K)],
            pltpu.VMEM_SHARED((nr_pad,), jnp.float32),
        ],
    )
    def kern(g2_hbm, w_hbm, g2_v, ones_v, slice_v, inv_v, w_ring, sems,
             cnt_sh):
        cid = lax.axis_index("c")
        sid = lax.axis_index("s")

        @pl.when(cid == 0)
        def _():
            for i in range(CH // L):
                ones_v[pl.ds(i * L, L)] = jnp.full((L,), 1.0, jnp.float32)

            def zfill(j, _):
                slice_v[pl.ds(j * L, L)] = jnp.zeros((L,), jnp.float32)
                return 0
            lax.fori_loop(0, inv_per_tile // L, zfill, 0)
            cbase = sid * inv_per_tile
            pltpu.sync_copy(slice_v, cnt_sh.at[pl.ds(cbase, inv_per_tile)])

            rbase = sid * per_tile
            pltpu.sync_copy(g2_hbm.at[pl.ds(rbase, per_tile)], g2_v)
            plsc.subcore_barrier()

            for s in range(K):
                pltpu.async_copy(ones_v, cnt_sh.at[g2_v.at[s, 0]], sems[s],
                                 add=True)

            def oct_body(q, _):
                for s in range(K):
                    pltpu.make_async_copy(
                        ones_v, cnt_sh.at[g2_v.at[s, 0]], sems[s]).wait()
                    pltpu.async_copy(
                        ones_v, cnt_sh.at[g2_v.at[q * K + s, 0]], sems[s],
                        add=True)
                return 0
            lax.fori_loop(1, n_oct, oct_body, 0)
            for s in range(K):
                pltpu.make_async_copy(
                    ones_v, cnt_sh.at[g2_v.at[s, 0]], sems[s]).wait()
            plsc.subcore_barrier()

            pltpu.sync_copy(cnt_sh.at[pl.ds(cbase, inv_per_tile)], slice_v)

            def inv_body(j, _):
                c = slice_v[pl.ds(j * L, L)]
                slice_v[pl.ds(j * L, L)] = 1.0 / jnp.maximum(c, 1.0)
                return 0
            lax.fori_loop(0, inv_per_tile // L, inv_body, 0)
            pltpu.sync_copy(slice_v, cnt_sh.at[pl.ds(cbase, inv_per_tile)])
            plsc.subcore_barrier()
            pltpu.sync_copy(cnt_sh, inv_v)

            def wchunk(c, s):
                for i in range(CH // L):
                    g2 = g2_v[c, 0, pl.ds(i * L, L)]
                    w_ring[s][pl.ds(i * L, L)] = plsc.load_gather(inv_v, [g2])
                pltpu.async_copy(w_ring[s], w_hbm.at[rbase + c], sems[s])

            for s in range(K):
                wchunk(s, s)

            def woct_body(q, _):
                for s in range(K):
                    pltpu.make_async_copy(
                        w_ring[s], w_hbm.at[0], sems[s]).wait()
                    wchunk(q * K + s, s)
                return 0
            lax.fori_loop(1, n_oct, woct_body, 0)
            for s in range(K):
                pltpu.make_async_copy(w_ring[s], w_hbm.at[0], sems[s]).wait()

    return kern


def _make_edge_pass(n, d, e_pad, acc_rows, ch, K, P):
    n_rows = e_pad // ch
    per_w = n_rows // 32
    g = d // L
    n_rounds = per_w // K
    assert per_w % K == 0 and P < K and K % 2 == 0 and (d // L) % 2 == 0
    rows_per_tile = acc_rows // 16
    full_tiles = n // rows_per_tile
    rem_rows = n - full_tiles * rows_per_tile
    zrows = min(ch, CH)
    mesh = plsc.VectorSubcoreMesh(core_axis_name="c", subcore_axis_name="s")

    @functools.partial(
        pl.kernel,
        mesh=mesh,
        out_type=jax.ShapeDtypeStruct((2, n, d), jnp.float32),
        compiler_params=pltpu.CompilerParams(
            needs_layout_passes=False, use_tc_tiling_on_sc=False),
        scratch_types=[
            pltpu.VMEM((per_w, 1, ch), jnp.int32),
            [pltpu.VMEM((1, ch), jnp.int32) for _ in range(K)],
            [pltpu.VMEM((ch,), jnp.float32) for _ in range(K)],
            [pltpu.VMEM((ch, d), jnp.bfloat16) for _ in range(K)],
            [pltpu.VMEM((ch, d), jnp.float32) for _ in range(2)],
            [pltpu.SemaphoreType.DMA for _ in range(K)],
            [pltpu.SemaphoreType.DMA for _ in range(2)],
            [pltpu.SemaphoreType.DMA for _ in range(K)],
            pltpu.VMEM_SHARED((acc_rows, d), jnp.float32),
        ],
    )
    def kern(tab_hbm, root_hbm, g1_hbm, dst_hbm, w_hbm, out_hbm,
             g1_v, dst_ring, w_ring, rows, stag, gsems, ssems, isems,
             acc_sh):
        cid = lax.axis_index("c")
        sid = lax.axis_index("s")
        wid = cid * 16 + sid
        rbase = wid * per_w

        pltpu.sync_copy(g1_hbm.at[pl.ds(rbase, per_w)], g1_v)

        def zrow(i, _):
            for k in range(g):
                stag[0][i, pl.ds(k * L, L)] = jnp.zeros((L,), jnp.float32)
            return 0
        lax.fori_loop(0, zrows, zrow, 0)

        base_row = sid * rows_per_tile

        @pl.when(jnp.logical_or(cid != 0, sid >= full_tiles))
        def _():
            for bidx in range(rows_per_tile // zrows):
                pltpu.sync_copy(
                    stag[0].at[pl.ds(0, zrows)],
                    acc_sh.at[pl.ds(base_row + bidx * zrows, zrows)])

        @pl.when(jnp.logical_and(cid == 0, sid < full_tiles))
        def _():
            pltpu.sync_copy(root_hbm.at[pl.ds(base_row, rows_per_tile)],
                            acc_sh.at[pl.ds(base_row, rows_per_tile)])

        if rem_rows > 0:
            @pl.when(jnp.logical_and(cid == 0, sid == full_tiles))
            def _():
                pltpu.sync_copy(root_hbm.at[pl.ds(base_row, rem_rows)],
                                acc_sh.at[pl.ds(base_row, rem_rows)])

        plsc.subcore_barrier()

        def issue_fetch(c, s):
            pltpu.async_copy(dst_hbm.at[rbase + c], dst_ring[s], isems[s])
            pltpu.async_copy(w_hbm.at[rbase + c], w_ring[s], isems[s])

        def wait_fetch(s):
            pltpu.make_async_copy(dst_hbm.at[0], dst_ring[s],
                                  isems[s]).wait()
            pltpu.make_async_copy(w_hbm.at[0], w_ring[s], isems[s]).wait()

        def issue_gather(c, s):
            pltpu.async_copy(tab_hbm.at[g1_v.at[c, 0]], rows[s], gsems[s])

        def wait_gather(s):
            pltpu.make_async_copy(tab_hbm.at[g1_v.at[0, 0]], rows[s],
                                  gsems[s]).wait()

        def issue_scatter(s, t):
            pltpu.async_copy(stag[t], acc_sh.at[dst_ring[s].at[0]], ssems[t],
                             add=True)

        def wait_scatter(s, t):
            pltpu.make_async_copy(stag[t], acc_sh.at[dst_ring[s].at[0]],
                                  ssems[t]).wait()

        def scale(s, t):
            def grp(i, _):
                wv = w_ring[s][pl.ds(i * L, L)]
                for j in range(L):
                    erow = i * L + j
                    w = wv[j]
                    for k in range(g // 2):
                        mb = rows[s][erow, pl.ds(k * 2 * L, 2 * L)]
                        a, b2 = plsc.unpack(
                            mb, format=plsc.PackFormat.INTERLEAVED)
                        stag[t][erow, pl.ds(k * 2 * L, L)] = a * w
                        stag[t][erow, pl.ds(k * 2 * L + L, L)] = b2 * w
                return 0
            lax.fori_loop(0, ch // L, grp, 0)

        def emit(c, s, t, first, skip_issue):
            if not first:
                wait_scatter((s + K - 1) % K, 1 - t)
            if not skip_issue:
                issue_fetch(c + P, (s + P) % K)
                issue_gather(c + P, (s + P) % K)
            wait_gather(s)
            wait_fetch(s)
            scale(s, t)
            issue_scatter(s, t)

        for s in range(P):
            issue_fetch(s, s)
            issue_gather(s, s)
        for s in range(K):
            emit(s, s, s % 2, s == 0, False)

        def round_body(q, _):
            for s in range(K):
                emit(q * K + s, s, s % 2, False, False)
            return 0
        lax.fori_loop(1, n_rounds - 1, round_body, 0)

        c0 = per_w - K
        for s in range(K):
            emit(c0 + s, s, s % 2, False, s + P >= K)
        wait_scatter(K - 1, (K - 1) % 2)

        plsc.subcore_barrier()

        @pl.when(sid < full_tiles)
        def _():
            pltpu.sync_copy(acc_sh.at[pl.ds(base_row, rows_per_tile)],
                            out_hbm.at[cid, pl.ds(base_row, rows_per_tile)])

        if rem_rows > 0:
            @pl.when(sid == full_tiles)
            def _():
                pltpu.sync_copy(acc_sh.at[pl.ds(base_row, rem_rows)],
                                out_hbm.at[cid, pl.ds(base_row, rem_rows)])

    return kern


def kernel(x, edge_index, edge_type, W1_rel, W1_root, b1, W2_rel, W2_root, b2):
    n, din = x.shape
    r, _, dh = W1_rel.shape
    do = W2_rel.shape[2]
    e = edge_index.shape[1]

    quantum = 640 * CH
    e_pad = ((e + quantum - 1) // quantum) * quantum
    pad = e_pad - e
    src = edge_index[0].astype(jnp.int32)
    dst = edge_index[1].astype(jnp.int32)
    typ = edge_type.astype(jnp.int32)
    if pad:
        src = jnp.concatenate([src, jnp.zeros((pad,), jnp.int32)])
        typ = jnp.concatenate([typ, jnp.zeros((pad,), jnp.int32)])
        dst = jnp.concatenate([dst, jnp.full((pad,), n, jnp.int32)])

    base_idx = typ * n + src
    g2 = (dst * r + typ).reshape(e_pad // CH, 1, CH)

    nr_pad = (((n + 1) * r + 16 * CH - 1) // (16 * CH)) * (16 * CH)
    acc_rows = ((n + 1 + 16 * CH - 1) // (16 * CH)) * (16 * CH)

    w_edge = _make_weights_kernel(e_pad, nr_pad)(g2)

    def _ileave(dd):
        p = []
        for b0 in range(0, dd, 2 * L):
            for k in range(L):
                p.extend([b0 + k, b0 + L + k])
        return np.array(p)

    w1p = W1_rel[:, :, _ileave(dh)]
    w2p = W2_rel[:, :, _ileave(do)]

    ch1 = 64
    tab1, root1 = _mm1(x, w1p, W1_root, b1)
    parts1 = _make_edge_pass(n, dh, e_pad, acc_rows, ch1, 4, 3)(
        tab1.reshape(n * r, dh), root1,
        base_idx.reshape(e_pad // ch1, 1, ch1),
        dst.reshape(e_pad // ch1, 1, ch1),
        w_edge.reshape(e_pad // ch1, ch1))

    ch2 = 128
    tab2, root2 = _mm2(parts1, w2p, W2_root, b2)
    parts2 = _make_edge_pass(n, do, e_pad, acc_rows, ch2, 4, 3)(
        tab2.reshape(n * r, do), root2,
        base_idx.reshape(e_pad // ch2, 1, ch2),
        dst.reshape(e_pad // ch2, 1, ch2),
        w_edge.reshape(e_pad // ch2, ch2))

    return _add_parts(parts2)
